# Initial kernel scaffold; baseline (speedup 1.0000x reference)
import jax
import jax.numpy as jnp
from jax.experimental import pallas as pl


def kernel(seq, seg, tok_embed, seg_embed, pos_embed):
    raise NotImplementedError("write your pallas kernel here")


# speedup vs baseline: 5.7315x; 5.7315x over previous
import functools

import jax
import jax.numpy as jnp
from jax import lax
from jax.experimental import pallas as pl
from jax.experimental.pallas import tpu as pltpu
from jax.experimental.pallas import tpu_sc as plsc

_B = 1024
_L = 512
_D = 128

_NC = 2
_NS = 16
_NW = _NC * _NS
_LANES = 16

_ROWS = _B * _L
_RPW = _ROWS // _NW
_CHUNK = 128
_STEPS = _RPW // _CHUNK


def _sc_body(tok_hbm, comb_hbm, seq_hbm, cidx_hbm, out_hbm,
             idx_tok, idx_comb, buf_a, buf_b, sem_a, sem_b):
    wid = lax.axis_index("s") * _NC + lax.axis_index("c")
    base = wid * _RPW

    def step(t, carry):
        off = base + t * _CHUNK
        pltpu.sync_copy(seq_hbm.at[pl.ds(off, _CHUNK)], idx_tok)
        pltpu.sync_copy(cidx_hbm.at[pl.ds(off, _CHUNK)], idx_comb)
        ga = pltpu.async_copy(tok_hbm.at[idx_tok], buf_a, sem_a)
        gb = pltpu.async_copy(comb_hbm.at[idx_comb], buf_b, sem_b)
        ga.wait()
        gb.wait()

        def row(r, c):
            for j in range(_D // _LANES):
                sl = pl.ds(j * _LANES, _LANES)
                plsc.addupdate(buf_a.at[r, sl], buf_b[r, sl])
            return c

        lax.fori_loop(0, _CHUNK, row, 0)
        pltpu.sync_copy(buf_a, out_hbm.at[pl.ds(off, _CHUNK)])
        return carry

    lax.fori_loop(0, _STEPS, step, 0)


def kernel(seq, seg, tok_embed, seg_embed, pos_embed):
    comb = (pos_embed[:, None, :] + seg_embed[None, :, :]).reshape(2 * _L, _D)
    cidx = (2 * jnp.arange(_L, dtype=jnp.int32)[None, :]
            + seg.astype(jnp.int32)).reshape(_ROWS)
    seq_flat = seq.astype(jnp.int32).reshape(_ROWS)

    mesh = plsc.VectorSubcoreMesh(core_axis_name="c", subcore_axis_name="s",
                                  num_cores=_NC, num_subcores=_NS)
    run = pl.kernel(
        _sc_body,
        out_type=jax.ShapeDtypeStruct((_ROWS, _D), jnp.float32),
        mesh=mesh,
        scratch_types=[
            pltpu.VMEM((_CHUNK,), jnp.int32),
            pltpu.VMEM((_CHUNK,), jnp.int32),
            pltpu.VMEM((_CHUNK, _D), jnp.float32),
            pltpu.VMEM((_CHUNK, _D), jnp.float32),
            pltpu.SemaphoreType.DMA,
            pltpu.SemaphoreType.DMA,
        ],
    )
    out = run(tok_embed, comb, seq_flat, cidx)
    return out.reshape(_B, _L, _D)


# speedup vs baseline: 9.6299x; 1.6802x over previous
import jax
import jax.numpy as jnp
from jax import lax
from jax.experimental import pallas as pl
from jax.experimental.pallas import tpu as pltpu
from jax.experimental.pallas import tpu_sc as plsc

_B = 1024
_L = 512
_D = 128

_NC = 2
_NS = 16
_NW = _NC * _NS
_LANES = 16

_ROWS = _B * _L
_RPW = _ROWS // _NW
_CHUNK = 64
_STEPS = _RPW // _CHUNK
_DEPTH = 4
_BYTES = _CHUNK * _D * 4


def _sc_body(tok_hbm, comb_hbm, seq_hbm, cidx_hbm, out_hbm,
             idx_tok, idx_comb,
             a0, a1, a2, a3, b0, b1, b2, b3,
             g0, g1, g2, g3, w0, w1, w2, w3):
    bufs_a = (a0, a1, a2, a3)
    bufs_b = (b0, b1, b2, b3)
    gsem = (g0, g1, g2, g3)
    wsem = (w0, w1, w2, w3)

    wid = lax.axis_index("s") * _NC + lax.axis_index("c")
    base = wid * _RPW

    pltpu.sync_copy(seq_hbm.at[wid], idx_tok)
    pltpu.sync_copy(cidx_hbm.at[wid], idx_comb)

    def fire(t, p):
        pltpu.async_copy(tok_hbm.at[idx_tok.at[t]], bufs_a[p], gsem[p])
        pltpu.async_copy(comb_hbm.at[idx_comb.at[t]], bufs_b[p], gsem[p])

    fire(0, 0)

    def outer(i, carry):
        for p in range(_DEPTH):
            t = _DEPTH * i + p
            tn = t + 1
            pn = (p + 1) % _DEPTH

            @pl.when(jnp.logical_and(t >= _DEPTH - 1, tn < _STEPS))
            def _():
                pltpu.make_async_copy(
                    bufs_a[pn], out_hbm.at[pl.ds(0, _CHUNK)], wsem[pn]).wait()

            @pl.when(tn < _STEPS)
            def _():
                fire(tn, pn)

            pltpu.make_async_copy(
                tok_hbm.at[idx_tok.at[0]], bufs_a[p], gsem[p]).wait()
            pltpu.make_async_copy(
                comb_hbm.at[idx_comb.at[0]], bufs_b[p], gsem[p]).wait()

            def row(r, c):
                for j in range(_D // _LANES):
                    sl = pl.ds(j * _LANES, _LANES)
                    plsc.addupdate(bufs_a[p].at[r, sl], bufs_b[p][r, sl])
                return c

            lax.fori_loop(0, _CHUNK, row, 0)

            off = base + t * _CHUNK
            pltpu.async_copy(bufs_a[p], out_hbm.at[pl.ds(off, _CHUNK)],
                             wsem[p])
        return carry

    lax.fori_loop(0, _STEPS // _DEPTH, outer, 0)

    for p in range(_DEPTH):
        pltpu.make_async_copy(
            bufs_a[p], out_hbm.at[pl.ds(0, _CHUNK)], wsem[p]).wait()


def kernel(seq, seg, tok_embed, seg_embed, pos_embed):
    comb = (pos_embed[:, None, :] + seg_embed[None, :, :]).reshape(2 * _L, _D)
    cidx = (2 * jnp.arange(_L, dtype=jnp.int32)[None, :]
            + seg.astype(jnp.int32)).reshape(_NW, _STEPS, _CHUNK)
    seq_flat = seq.astype(jnp.int32).reshape(_NW, _STEPS, _CHUNK)

    mesh = plsc.VectorSubcoreMesh(core_axis_name="c", subcore_axis_name="s",
                                  num_cores=_NC, num_subcores=_NS)
    run = pl.kernel(
        _sc_body,
        out_type=jax.ShapeDtypeStruct((_ROWS, _D), jnp.float32),
        mesh=mesh,
        scratch_types=(
            [pltpu.VMEM((_STEPS, _CHUNK), jnp.int32)] * 2
            + [pltpu.VMEM((_CHUNK, _D), jnp.float32)] * (2 * _DEPTH)
            + [pltpu.SemaphoreType.DMA] * (2 * _DEPTH)
        ),
    )
    out = run(tok_embed, comb, seq_flat, cidx)
    return out.reshape(_B, _L, _D)
